# baseline (speedup 1.0000x reference)
import functools

import jax
import jax.numpy as jnp
from jax import lax
from jax.experimental import pallas as pl
from jax.experimental.pallas import tpu as pltpu
from jax.experimental.pallas import tpu_sc as plsc

BATCH = 16384
HIST_LEN = 50
EMBED_DIM = 32

SUB = 128
SG = BATCH // SUB
UNITS = HIST_LEN * SG
NC, NS = 2, 16
NW = NC * NS
UPW = UNITS // NW
DG = EMBED_DIM // 8

_MESH = plsc.VectorSubcoreMesh(core_axis_name="c", subcore_axis_name="s")


@functools.partial(
    pl.kernel,
    mesh=_MESH,
    out_type=jax.ShapeDtypeStruct((HIST_LEN, DG, SG, 8, SUB), jnp.float32),
    compiler_params=pltpu.CompilerParams(
        use_tc_tiling_on_sc=False, needs_layout_passes=False),
    scratch_types=[
        pltpu.VMEM((UPW, SUB), jnp.int32),
        pltpu.VMEM((4, SUB, EMBED_DIM), jnp.float32),
        pltpu.VMEM((2, DG, 8, SUB), jnp.float32),
        pltpu.SemaphoreType.DMA((4,)),
        pltpu.SemaphoreType.DMA((2,)),
    ],
)
def _gather_sc(table_hbm, idx_hbm, out_hbm, idx_all, rows_g, rows_t, sem_g, sem_o):
    wid = lax.axis_index("s") * NC + lax.axis_index("c")
    u0 = wid * UPW

    pltpu.sync_copy(idx_hbm.at[pl.ds(u0, UPW)], idx_all)

    def gather(t):
        q = lax.rem(t, 4)
        return pltpu.make_async_copy(
            table_hbm.at[idx_all.at[t]], rows_g.at[q], sem_g.at[q])

    def store(t, dg):
        u = u0 + t
        h = lax.div(u, SG)
        sg = lax.rem(u, SG)
        q = lax.rem(t, 2)
        return pltpu.make_async_copy(
            rows_t.at[q, dg], out_hbm.at[h, dg, sg], sem_o.at[q])

    riota = [lax.iota(jnp.int32, 16) + 16 * k for k in range(8)]

    gather(0).start()
    gather(1).start()
    gather(2).start()

    def unit(t, carry):
        q4 = lax.rem(t, 4)
        q2 = lax.rem(t, 2)

        @pl.when(t >= 2)
        def _():
            for dg in range(DG):
                store(t - 2, dg).wait()

        gather(t).wait()

        @pl.when(t + 3 < UPW)
        def _():
            gather(t + 3).start()

        src = rows_g.at[q4]

        for d in range(EMBED_DIM):
            cvec = jnp.full((16,), d, jnp.int32)
            for k in range(8):
                v = plsc.load_gather(src, [riota[k], cvec])
                rows_t[q2, d // 8, d % 8, pl.ds(16 * k, 16)] = v

        for dg in range(DG):
            store(t, dg).start()
        return carry

    lax.fori_loop(0, UPW, unit, 0)

    for dg in range(DG):
        store(UPW - 2, dg).wait()
        store(UPW - 1, dg).wait()


def kernel(nouns_idx_tensor, histwords_embeddings):
    idx = nouns_idx_tensor.astype(jnp.int32).T.reshape(UNITS, SUB)
    out5 = _gather_sc(histwords_embeddings, idx)
    return out5.transpose(2, 4, 0, 1, 3).reshape(BATCH, HIST_LEN, EMBED_DIM)


# speedup vs baseline: 1.3978x; 1.3978x over previous
import functools

import jax
import jax.numpy as jnp
from jax import lax
from jax.experimental import pallas as pl
from jax.experimental.pallas import tpu as pltpu
from jax.experimental.pallas import tpu_sc as plsc

BATCH = 16384
HIST_LEN = 50
EMBED_DIM = 32

SUB = 128
SG = BATCH // SUB
UNITS = HIST_LEN * SG
NC, NS = 2, 16
NW = NC * NS
UPW = UNITS // NW
DG = EMBED_DIM // 8

_MESH = plsc.VectorSubcoreMesh(core_axis_name="c", subcore_axis_name="s")


@functools.partial(
    pl.kernel,
    mesh=_MESH,
    out_type=jax.ShapeDtypeStruct((HIST_LEN, DG, SG, 8, SUB), jnp.float32),
    compiler_params=pltpu.CompilerParams(
        use_tc_tiling_on_sc=False, needs_layout_passes=False),
    scratch_types=[
        pltpu.VMEM((UPW, SUB), jnp.int32),
        pltpu.VMEM((4, SUB, EMBED_DIM), jnp.float32),
        pltpu.VMEM((2, EMBED_DIM, SUB), jnp.float32),
        pltpu.SemaphoreType.DMA((4,)),
        pltpu.SemaphoreType.DMA((2,)),
    ],
)
def _gather_sc(table_hbm, idx_hbm, out_hbm, idx_all, rows_g, rows_t, sem_g, sem_o):
    wid = lax.axis_index("s") * NC + lax.axis_index("c")
    u0 = wid * UPW

    pltpu.sync_copy(idx_hbm.at[pl.ds(u0, UPW)], idx_all)

    def gather(t):
        q = lax.rem(t, 4)
        return pltpu.make_async_copy(
            table_hbm.at[idx_all.at[t]], rows_g.at[q], sem_g.at[q])

    def store(t, dg):
        u = u0 + t
        h = lax.div(u, SG)
        sg = lax.rem(u, SG)
        q = lax.rem(t, 2)
        return pltpu.make_async_copy(
            rows_t.at[q, pl.ds(dg * 8, 8)], out_hbm.at[h, dg, sg], sem_o.at[q])

    lanevec = lax.iota(jnp.int32, 16)
    riota = [lanevec + 16 * k for k in range(8)]

    gather(0).start()
    gather(1).start()
    gather(2).start()

    def unit(t, carry):
        q4 = lax.rem(t, 4)
        q2 = lax.rem(t, 2)

        @pl.when(t >= 2)
        def _():
            for dg in range(DG):
                store(t - 2, dg).wait()

        gather(t).wait()

        @pl.when(t + 3 < UPW)
        def _():
            gather(t + 3).start()

        src = rows_g.at[q4]
        qvec = jnp.full((16,), 0, jnp.int32) + q2

        for d0 in range(EMBED_DIM):
            cvec = (lanevec + d0) & (EMBED_DIM - 1)
            for k in range(8):
                v = plsc.load_gather(src, [riota[k], cvec])
                plsc.store_scatter(rows_t, [qvec, cvec, riota[k]], v)

        for dg in range(DG):
            store(t, dg).start()
        return carry

    lax.fori_loop(0, UPW, unit, 0)

    for dg in range(DG):
        store(UPW - 2, dg).wait()
        store(UPW - 1, dg).wait()


def kernel(nouns_idx_tensor, histwords_embeddings):
    idx = nouns_idx_tensor.astype(jnp.int32).T.reshape(UNITS, SUB)
    out5 = _gather_sc(histwords_embeddings, idx)
    return out5.transpose(2, 4, 0, 1, 3).reshape(BATCH, HIST_LEN, EMBED_DIM)
